# Initial kernel scaffold; baseline (speedup 1.0000x reference)
import jax
import jax.numpy as jnp
from jax.experimental import pallas as pl


def kernel(h_CLS, H_genes_pert, perturbation_indices, batch_assignment, W1, b1, W2, b2):
    raise NotImplementedError("write your pallas kernel here")


# speedup vs baseline: 3.5122x; 3.5122x over previous
import jax
import jax.numpy as jnp
from jax import lax
from jax.experimental import pallas as pl
from jax.experimental.pallas import tpu as pltpu
from jax.experimental.pallas import tpu_sc as plsc

_NC = 2
_NS = 16
_NW = _NC * _NS
_LANES = 16


def _sc_gather(h_flat, pert_idx, batch_assignment, *, N, D, P):
    K = P // _NW

    mesh = plsc.VectorSubcoreMesh(core_axis_name="c", subcore_axis_name="s")

    def body(h_hbm, pi_hbm, ba_hbm, out_hbm, pi_v, ba_v, flat_v, rows_v, sem):
        cid = lax.axis_index("c")
        sid = lax.axis_index("s")
        wid = sid * _NC + cid
        base = wid * K

        pltpu.sync_copy(pi_hbm.at[pl.ds(base, K)], pi_v)
        pltpu.sync_copy(ba_hbm.at[pl.ds(base, K)], ba_v)

        for c in range(K // _LANES):
            s = pl.ds(c * _LANES, _LANES)
            flat_v[s] = ba_v[s] * N + pi_v[s]

        pltpu.async_copy(h_hbm.at[flat_v], rows_v, sem).wait()

        pltpu.sync_copy(rows_v, out_hbm.at[pl.ds(base, K)])

    call = pl.kernel(
        body,
        out_type=jax.ShapeDtypeStruct((P, D), jnp.float32),
        mesh=mesh,
        scratch_types=[
            pltpu.VMEM((K,), jnp.int32),
            pltpu.VMEM((K,), jnp.int32),
            pltpu.VMEM((K,), jnp.int32),
            pltpu.VMEM((K, D), jnp.float32),
            pltpu.SemaphoreType.DMA,
        ],
    )
    return call(h_flat, pert_idx, batch_assignment)


def _tc_head(gathered, ba_row, hcls_row, W1, b1_row, w2_row, b2_11, *, B, D, P):
    def body(g_ref, ba_ref, hcls_ref, w1_ref, b1_ref, w2_ref, b2_ref, out_ref):
        ba = ba_ref[...]
        bid = lax.broadcasted_iota(jnp.int32, (B, P), 0)
        onehot = (bid == ba).astype(jnp.float32)
        sums = jnp.dot(onehot, g_ref[...],
                       preferred_element_type=jnp.float32)
        counts = jnp.sum(onehot, axis=1, keepdims=True)
        z = sums / jnp.maximum(counts, 1.0)
        h1 = jnp.dot(hcls_ref[...], w1_ref[0:D, :],
                     preferred_element_type=jnp.float32)
        h2 = jnp.dot(z, w1_ref[D:2 * D, :],
                     preferred_element_type=jnp.float32)
        hidden = jnp.maximum(h1 + h2 + b1_ref[...], 0.0)
        pred = jnp.sum(hidden * w2_ref[...], axis=1, keepdims=True)
        out_ref[...] = pred + b2_ref[...]

    return pl.pallas_call(
        body,
        out_shape=jax.ShapeDtypeStruct((B, 1), jnp.float32),
    )(gathered, ba_row, hcls_row, W1, b1_row, w2_row, b2_11)


def kernel(h_CLS, H_genes_pert, perturbation_indices, batch_assignment,
           W1, b1, W2, b2):
    B, N, D = H_genes_pert.shape
    P = perturbation_indices.shape[0]

    h_flat = H_genes_pert.reshape(B * N, D)

    gathered = _sc_gather(h_flat, perturbation_indices, batch_assignment,
                          N=N, D=D, P=P)

    return _tc_head(gathered,
                    batch_assignment.reshape(1, P),
                    h_CLS.reshape(1, D),
                    W1,
                    b1.reshape(1, D),
                    W2.reshape(1, D),
                    b2.reshape(1, 1),
                    B=B, D=D, P=P)
